# baseline (speedup 1.0000x reference)
import functools

import jax
import jax.numpy as jnp
from jax import lax
from jax.experimental import pallas as pl
from jax.experimental.pallas import tpu as pltpu
from jax.experimental.pallas import tpu_sc as plsc

CHUNK = 128


def _make_sc_gather(V, D, B):
    info = plsc.get_sparse_core_info()
    NW = info.num_cores * info.num_subcores
    assert B % NW == 0
    b_per_w = B // NW
    assert b_per_w % CHUNK == 0
    n_chunks = b_per_w // CHUNK
    mesh = plsc.VectorSubcoreMesh(core_axis_name="c", subcore_axis_name="s")

    @functools.partial(
        pl.kernel,
        mesh=mesh,
        out_type=jax.ShapeDtypeStruct((B, D), jnp.float32),
        scratch_types=[
            pltpu.VMEM((n_chunks, CHUNK), jnp.int32),
            pltpu.VMEM((b_per_w, D), jnp.float32),
            pltpu.SemaphoreType.DMA,
        ],
        compiler_params=pltpu.CompilerParams(use_tc_tiling_on_sc=False),
    )
    def sc_gather(table_hbm, idx_hbm, out_hbm, idx_v, rows_v, sem):
        wid = lax.axis_index("s") * info.num_cores + lax.axis_index("c")
        base = wid * b_per_w
        pltpu.sync_copy(idx_hbm.at[wid], idx_v)
        copies = []
        for j in range(n_chunks):
            copies.append(
                pltpu.make_async_copy(
                    table_hbm.at[idx_v.at[j]],
                    rows_v.at[pl.ds(j * CHUNK, CHUNK)],
                    sem,
                )
            )
        for c in copies:
            c.start()
        for c in copies:
            c.wait()
        pltpu.sync_copy(rows_v, out_hbm.at[pl.ds(base, b_per_w)])

    return sc_gather


def kernel(labels, table):
    V, D = table.shape
    (B,) = labels.shape
    info = plsc.get_sparse_core_info()
    NW = info.num_cores * info.num_subcores
    fn = _make_sc_gather(V, D, B)
    idx3 = labels.astype(jnp.int32).reshape(NW, (B // NW) // CHUNK, CHUNK)
    return fn(table, idx3)


# speedup vs baseline: 2.9164x; 2.9164x over previous
import functools

import jax
import jax.numpy as jnp
from jax import lax
from jax.experimental import pallas as pl
from jax.experimental.pallas import tpu as pltpu
from jax.experimental.pallas import tpu_sc as plsc

L = 16
D = 32
WIN_BLKS = 8
WIN_COLS = WIN_BLKS * 128
VALS_CAP = 2048
WL_CAP = 2048


def _make_k1(V, B, NC, NS):
    NW = NC * NS
    nblk = (V + 127) // 128
    last_blk = nblk - 1
    tail_cols = V - last_blk * 128
    max_dma_blk = (V - WIN_COLS) // 128
    n_stage = D * B
    n_win = (nblk // NW) // WIN_BLKS + 1
    mesh = plsc.VectorSubcoreMesh(core_axis_name="c", subcore_axis_name="s")

    @functools.partial(
        pl.kernel,
        mesh=mesh,
        out_type=jax.ShapeDtypeStruct((NC, D, B), jnp.float32),
        scratch_types=[
            pltpu.VMEM((D, WIN_COLS), jnp.float32),
            pltpu.VMEM((D, tail_cols), jnp.float32),
            pltpu.VMEM((2048,), jnp.int32),
            pltpu.VMEM((B,), jnp.int32),
            pltpu.VMEM((B,), jnp.int32),
            pltpu.VMEM((WL_CAP,), jnp.int32),
            pltpu.VMEM((WL_CAP,), jnp.int32),
            pltpu.VMEM((VALS_CAP,), jnp.float32),
            pltpu.VMEM((VALS_CAP,), jnp.int32),
            pltpu.VMEM((8, 2048), jnp.float32),
            pltpu.VMEM_SHARED((n_stage + L,), jnp.float32),
        ],
        compiler_params=pltpu.CompilerParams(needs_layout_passes=False),
    )
    def k1(tab_hbm, tail_hbm, lab_hbm, out_hbm, win_v, tail_v, stage_v,
           own_r, own_b, wl_r, wl_b, vals_v, oidx_v, drain_v, sh_stage):
        c = lax.axis_index("c")
        s = lax.axis_index("s")
        w = c * NS + s
        iota = lax.iota(jnp.int32, L)
        dump_vec = jnp.full((L,), n_stage, jnp.int32) + iota

        zv = jnp.zeros((L,), jnp.float32)
        for i in range(VALS_CAP // L):
            vals_v[pl.ds(i * L, L)] = zv
        share = n_stage // NS
        for i in range(share // VALS_CAP):
            pltpu.sync_copy(
                vals_v, sh_stage.at[pl.ds(s * share + i * VALS_CAP, VALS_CAP)]
            )
        for i in range(VALS_CAP // L):
            oidx_v[pl.ds(i * L, L)] = dump_vec
        plsc.subcore_barrier()

        b0 = (w * nblk) // NW
        e_all = ((w + 1) * nblk) // NW
        e_reg = jnp.minimum(e_all, last_blk)

        def scan_chunk(k, cur, base):
            r = stage_v[pl.ds(k * L, L)]
            blk = lax.shift_right_logical(r, 7)
            m = (blk >= b0) & (blk < e_all)
            mi = m.astype(jnp.int32)
            pos = cur + plsc.cumsum(mi) - mi
            cnt = lax.reduce_sum(mi, axes=(0,))
            plsc.store_scatter(own_r, [pos], r, mask=m)
            bpos = jnp.full((L,), base + k * L, jnp.int32) + iota
            plsc.store_scatter(own_b, [pos], bpos, mask=m)
            return cur + cnt

        cur = jnp.int32(0)
        for t in range(B // 2048):
            pltpu.sync_copy(lab_hbm.at[pl.ds(t * 2048, 2048)], stage_v)
            cur = lax.fori_loop(
                0, 2048 // L,
                lambda k, a, _t=t: scan_chunk(k, a, _t * 2048),
                cur,
            )

        n_own_chunks = (cur + L - 1) // L

        def extract(win_ref, col0, wb0, wb1, cursor):
            def do_ext(wcur, csr):
                def ext_chunk(j, csr):
                    lanes = (j * L + iota) < wcur
                    roff = wl_r[pl.ds(j * L, L)]
                    bv = wl_b[pl.ds(j * L, L)]
                    roff = jnp.where(lanes, roff, 0)

                    do_flush = csr + D * L > VALS_CAP

                    @pl.when(do_flush)
                    def _():
                        pltpu.sync_copy(vals_v, sh_stage.at[oidx_v], add=True)
                        for i in range(VALS_CAP // L):
                            oidx_v[pl.ds(i * L, L)] = dump_vec

                    csr = jnp.where(do_flush, 0, csr)
                    for cc in range(D):
                        cvec = jnp.full((L,), cc, jnp.int32)
                        val = plsc.load_gather(win_ref, [cvec, roff])
                        oi = jnp.where(lanes, cc * B + bv, dump_vec)
                        vals_v[pl.ds(csr + cc * L, L)] = val
                        oidx_v[pl.ds(csr + cc * L, L)] = oi
                    return csr + D * L

                n_wl = (wcur + L - 1) // L
                return lax.fori_loop(0, n_wl, ext_chunk, csr)

            def wl_chunk(k, st):
                wcur, csr = st
                r = own_r[pl.ds(k * L, L)]
                lanes = (k * L + iota) < cur
                blk = lax.shift_right_logical(r, 7)
                m = lanes & (blk >= wb0) & (blk < wb1)
                mi = m.astype(jnp.int32)
                cnt = lax.reduce_sum(mi, axes=(0,))
                full = wcur + cnt > WL_CAP
                csr = lax.cond(
                    full, lambda: do_ext(wcur, csr), lambda: csr
                )
                wcur = jnp.where(full, 0, wcur)
                pos = wcur + plsc.cumsum(mi) - mi
                bv = own_b[pl.ds(k * L, L)]
                plsc.store_scatter(wl_r, [pos], r - col0, mask=m)
                plsc.store_scatter(wl_b, [pos], bv, mask=m)
                return wcur + cnt, csr

            wcur, csr = lax.fori_loop(
                0, n_own_chunks, wl_chunk, (jnp.int32(0), cursor)
            )
            return do_ext(wcur, csr)

        def win_body(i, cursor):
            wb0 = b0 + i * WIN_BLKS
            wb1 = jnp.minimum(wb0 + WIN_BLKS, e_reg)
            dma_blk = jnp.minimum(wb0, max_dma_blk)
            col0 = pl.multiple_of(dma_blk * 128, 128)

            @pl.when(wb0 < wb1)
            def _():
                pltpu.sync_copy(
                    tab_hbm.at[:, pl.ds(col0, WIN_COLS)], win_v
                )

            return jnp.where(
                wb0 < wb1, extract(win_v, col0, wb0, wb1, cursor), cursor
            )

        cursor = lax.fori_loop(0, n_win, win_body, jnp.int32(0))

        @pl.when(w == NW - 1)
        def _():
            pltpu.sync_copy(tail_hbm, tail_v)

        tail_cursor = jnp.where(
            w == NW - 1,
            extract(tail_v, last_blk * 128, last_blk, last_blk + 1, cursor),
            cursor,
        )
        del tail_cursor
        pltpu.sync_copy(vals_v, sh_stage.at[oidx_v], add=True)

        plsc.subcore_barrier()

        band = s // 4
        bandq = s % 4
        for half in range(B // 4 // 2048):
            for rr in range(8):
                row = (s // 4) * 8 + rr
                off = row * B + bandq * (B // 4) + half * 2048
                pltpu.sync_copy(
                    sh_stage.at[pl.ds(off, 2048)], drain_v.at[rr]
                )
            pltpu.sync_copy(
                drain_v,
                out_hbm.at[
                    c,
                    pl.ds((s // 4) * 8, 8),
                    pl.ds(bandq * (B // 4) + half * 2048, 2048),
                ],
            )

    return k1


def _make_k2(B, NC, NS):
    NW = NC * NS
    bs = B // NW
    mesh = plsc.VectorSubcoreMesh(core_axis_name="c", subcore_axis_name="s")

    @functools.partial(
        pl.kernel,
        mesh=mesh,
        out_type=jax.ShapeDtypeStruct((D, B), jnp.float32),
        scratch_types=[
            pltpu.VMEM((D, bs), jnp.float32),
            pltpu.VMEM((D, bs), jnp.float32),
        ],
    )
    def k2(inter_hbm, out_hbm, p0_v, p1_v):
        c = lax.axis_index("c")
        s = lax.axis_index("s")
        w = c * NS + s
        base = w * bs
        pltpu.sync_copy(inter_hbm.at[0, :, pl.ds(base, bs)], p0_v)
        pltpu.sync_copy(inter_hbm.at[1, :, pl.ds(base, bs)], p1_v)

        def row_body(r, _):
            def chunk(k, _):
                a = p0_v[r, pl.ds(k * L, L)]
                b = p1_v[r, pl.ds(k * L, L)]
                p0_v[r, pl.ds(k * L, L)] = a + b
                return 0

            return lax.fori_loop(0, bs // L, chunk, 0)

        lax.fori_loop(0, D, row_body, 0)
        pltpu.sync_copy(p0_v, out_hbm.at[:, pl.ds(base, bs)])

    return k2


def kernel(labels, table):
    V, Dd = table.shape
    (B,) = labels.shape
    info = plsc.get_sparse_core_info()
    NC, NS = info.num_cores, info.num_subcores
    tableT = table.T
    last_blk = (V - 1) // 128
    tailT = table[last_blk * 128:].T
    inter = _make_k1(V, B, NC, NS)(tableT, tailT, labels.astype(jnp.int32))
    outT = _make_k2(B, NC, NS)(inter)
    return outT.T
